# Initial kernel scaffold; baseline (speedup 1.0000x reference)
import jax
import jax.numpy as jnp
from jax.experimental import pallas as pl


def kernel(x, edge_index, W11, b11, Wout, bout):
    raise NotImplementedError("write your pallas kernel here")


# speedup vs baseline: 7.2708x; 7.2708x over previous
import functools

import jax
import jax.numpy as jnp
from jax import lax
from jax.experimental import pallas as pl
from jax.experimental.pallas import tpu as pltpu
from jax.experimental.pallas import tpu_sc as plsc

N = 10000
E = 320000
HIDDEN = 128
CLASSES = 40

D_PAD = 48
N_PAD = 10112
NC = 2
NS = 16
NW = NC * NS
LANE = 128
EW = -(-E // NW)
CH = -(-EW // LANE)
E_PAD = NW * CH * LANE
STRIPE = N_PAD // NS


def _mm_body(w11_ref, wout_ref, m_ref):
    m_ref[...] = lax.dot_general(
        w11_ref[...], wout_ref[...],
        dimension_numbers=(((0,), (1,)), ((), ())),
        preferred_element_type=jnp.float32,
    )


def _make_table(w11p, woutp):
    return pl.pallas_call(
        _mm_body,
        out_shape=jax.ShapeDtypeStruct((N_PAD, D_PAD), jnp.float32),
    )(w11p, woutp)


def _sc_body(m_hbm, col_hbm, row_hbm, zero_hbm, out_hbm,
             col_v, row_v, buf, acc, sem):
    cid = lax.axis_index("c")
    sid = lax.axis_index("s")
    pltpu.sync_copy(col_hbm.at[cid, sid], col_v)
    pltpu.sync_copy(row_hbm.at[cid, sid], row_v)
    pltpu.sync_copy(zero_hbm.at[pl.ds(sid * STRIPE, STRIPE)],
                    acc.at[pl.ds(sid * STRIPE, STRIPE)])
    plsc.subcore_barrier()

    def body(j, carry):
        pltpu.async_copy(m_hbm.at[col_v.at[j]], buf, sem).wait()
        pltpu.sync_copy(buf, acc.at[row_v.at[j]], add=True)
        return carry

    lax.fori_loop(0, CH, body, 0)
    plsc.subcore_barrier()
    pltpu.sync_copy(acc.at[pl.ds(sid * STRIPE, STRIPE)],
                    out_hbm.at[cid, pl.ds(sid * STRIPE, STRIPE)])


@functools.cache
def _sc_scatter():
    return pl.kernel(
        _sc_body,
        out_type=jax.ShapeDtypeStruct((NC, N_PAD, D_PAD), jnp.float32),
        mesh=plsc.VectorSubcoreMesh(core_axis_name="c", subcore_axis_name="s",
                                    num_cores=NC, num_subcores=NS),
        scratch_types=[
            pltpu.VMEM((CH, LANE), jnp.int32),
            pltpu.VMEM((CH, LANE), jnp.int32),
            pltpu.VMEM((LANE, D_PAD), jnp.float32),
            pltpu.VMEM_SHARED((N_PAD, D_PAD), jnp.float32),
            pltpu.SemaphoreType.DMA,
        ],
        compiler_params=pltpu.CompilerParams(use_tc_tiling_on_sc=False),
    )


def _softmax_body(p_ref, b11_ref, wout_ref, bout_ref, ls_ref, h_ref):
    bias = lax.dot_general(
        b11_ref[...], wout_ref[...],
        dimension_numbers=(((1,), (1,)), ((), ())),
        preferred_element_type=jnp.float32,
    ) + bout_ref[...]
    h = p_ref[0] + p_ref[1] + bias
    col = lax.broadcasted_iota(jnp.int32, (N_PAD, D_PAD), 1)
    hm = jnp.where(col < CLASSES, h, -jnp.inf)
    m = jnp.max(hm, axis=1, keepdims=True)
    lse = jnp.log(jnp.sum(jnp.exp(hm - m), axis=1, keepdims=True)) + m
    ls_ref[...] = h - lse
    h_ref[...] = h


def _finalize(partials, b11r, woutp, boutr):
    return pl.pallas_call(
        _softmax_body,
        out_shape=(jax.ShapeDtypeStruct((N_PAD, D_PAD), jnp.float32),
                   jax.ShapeDtypeStruct((N_PAD, D_PAD), jnp.float32)),
    )(partials, b11r, woutp, boutr)


def kernel(x, edge_index, W11, b11, Wout, bout):
    del x
    row = edge_index[0]
    col = edge_index[1]
    pad = E_PAD - E
    rowp = jnp.concatenate([row, jnp.full((pad,), N, jnp.int32)])
    colp = jnp.concatenate([col, jnp.zeros((pad,), jnp.int32)])
    rowp = rowp.reshape(NC, NS, CH, LANE)
    colp = colp.reshape(NC, NS, CH, LANE)

    w11p = jnp.pad(W11, ((0, 0), (0, N_PAD - N)))
    woutp = jnp.pad(Wout, ((0, D_PAD - CLASSES), (0, 0)))
    boutr = jnp.pad(bout, (0, D_PAD - CLASSES)).reshape(1, D_PAD)
    b11r = b11.reshape(1, HIDDEN)
    zero = jnp.zeros((N_PAD, D_PAD), jnp.float32)

    m_table = _make_table(w11p, woutp)
    partials = _sc_scatter()(m_table, colp, rowp, zero)
    ls, h = _finalize(partials, b11r, woutp, boutr)
    return (ls[:N, :CLASSES], h[:N, :CLASSES])


# speedup vs baseline: 8.8191x; 1.2129x over previous
import functools

import jax
import jax.numpy as jnp
from jax import lax
from jax.experimental import pallas as pl
from jax.experimental.pallas import tpu as pltpu
from jax.experimental.pallas import tpu_sc as plsc

N = 10000
E = 320000
HIDDEN = 128
CLASSES = 40

D_PAD = 48
N_PAD = 10112
NC = 2
NS = 16
NW = NC * NS
LANE = 128
EW = -(-E // NW)
CH = -(-EW // LANE)
E_PAD = NW * CH * LANE
STRIPE = N_PAD // NS


def _mm_body(w11_ref, wout_ref, m_ref):
    m_ref[...] = lax.dot_general(
        w11_ref[...], wout_ref[...],
        dimension_numbers=(((0,), (1,)), ((), ())),
        preferred_element_type=jnp.float32,
    )


def _make_table(w11p, woutp):
    return pl.pallas_call(
        _mm_body,
        out_shape=jax.ShapeDtypeStruct((N_PAD, D_PAD), jnp.float32),
    )(w11p, woutp)


def _sc_body(m_hbm, col_hbm, row_hbm, zero_hbm, out_hbm,
             col_v, row_v, buf0, buf1, acc, sem0, sem1):
    cid = lax.axis_index("c")
    sid = lax.axis_index("s")
    pltpu.sync_copy(col_hbm.at[cid, sid], col_v)
    pltpu.sync_copy(row_hbm.at[cid, sid], row_v)
    pltpu.sync_copy(zero_hbm.at[pl.ds(sid * STRIPE, STRIPE)],
                    acc.at[pl.ds(sid * STRIPE, STRIPE)])
    plsc.subcore_barrier()

    bufs = (buf0, buf1)
    sems = (sem0, sem1)
    pltpu.async_copy(m_hbm.at[col_v.at[0]], buf0, sem0)
    pltpu.async_copy(m_hbm.at[col_v.at[1]], buf1, sem1)

    def body(j, carry):
        for b in range(2):
            c = 2 * j + b
            if 2 * ((CH + 1) // 2) - 2 + b < CH:
                pltpu.make_async_copy(m_hbm.at[pl.ds(0, LANE)],
                                      bufs[b], sems[b]).wait()
                pltpu.sync_copy(bufs[b], acc.at[row_v.at[c]], add=True)

                @pl.when(c + 2 < CH)
                def _():
                    pltpu.async_copy(m_hbm.at[col_v.at[c + 2]],
                                     bufs[b], sems[b])
            else:

                @pl.when(c < CH)
                def _():
                    pltpu.make_async_copy(m_hbm.at[pl.ds(0, LANE)],
                                          bufs[b], sems[b]).wait()
                    pltpu.sync_copy(bufs[b], acc.at[row_v.at[c]], add=True)

                    @pl.when(c + 2 < CH)
                    def _():
                        pltpu.async_copy(m_hbm.at[col_v.at[c + 2]],
                                         bufs[b], sems[b])
        return carry

    lax.fori_loop(0, (CH + 1) // 2, body, 0)
    plsc.subcore_barrier()
    pltpu.sync_copy(acc.at[pl.ds(sid * STRIPE, STRIPE)],
                    out_hbm.at[cid, pl.ds(sid * STRIPE, STRIPE)])


@functools.cache
def _sc_scatter():
    return pl.kernel(
        _sc_body,
        out_type=jax.ShapeDtypeStruct((NC, N_PAD, D_PAD), jnp.float32),
        mesh=plsc.VectorSubcoreMesh(core_axis_name="c", subcore_axis_name="s",
                                    num_cores=NC, num_subcores=NS),
        scratch_types=[
            pltpu.VMEM((CH, LANE), jnp.int32),
            pltpu.VMEM((CH, LANE), jnp.int32),
            pltpu.VMEM((LANE, D_PAD), jnp.float32),
            pltpu.VMEM((LANE, D_PAD), jnp.float32),
            pltpu.VMEM_SHARED((N_PAD, D_PAD), jnp.float32),
            pltpu.SemaphoreType.DMA,
            pltpu.SemaphoreType.DMA,
        ],
        compiler_params=pltpu.CompilerParams(use_tc_tiling_on_sc=False),
    )


def _softmax_body(p_ref, b11_ref, wout_ref, bout_ref, ls_ref, h_ref):
    bias = lax.dot_general(
        b11_ref[...], wout_ref[...],
        dimension_numbers=(((1,), (1,)), ((), ())),
        preferred_element_type=jnp.float32,
    ) + bout_ref[...]
    h = p_ref[0] + p_ref[1] + bias
    col = lax.broadcasted_iota(jnp.int32, (N_PAD, D_PAD), 1)
    hm = jnp.where(col < CLASSES, h, -jnp.inf)
    m = jnp.max(hm, axis=1, keepdims=True)
    lse = jnp.log(jnp.sum(jnp.exp(hm - m), axis=1, keepdims=True)) + m
    ls_ref[...] = h - lse
    h_ref[...] = h


def _finalize(partials, b11r, woutp, boutr):
    return pl.pallas_call(
        _softmax_body,
        out_shape=(jax.ShapeDtypeStruct((N_PAD, D_PAD), jnp.float32),
                   jax.ShapeDtypeStruct((N_PAD, D_PAD), jnp.float32)),
    )(partials, b11r, woutp, boutr)


def kernel(x, edge_index, W11, b11, Wout, bout):
    del x
    row = edge_index[0]
    col = edge_index[1]
    pad = E_PAD - E
    rowp = jnp.concatenate([row, jnp.full((pad,), N, jnp.int32)])
    colp = jnp.concatenate([col, jnp.zeros((pad,), jnp.int32)])
    rowp = rowp.reshape(NC, NS, CH, LANE)
    colp = colp.reshape(NC, NS, CH, LANE)

    w11p = jnp.pad(W11, ((0, 0), (0, N_PAD - N)))
    woutp = jnp.pad(Wout, ((0, D_PAD - CLASSES), (0, 0)))
    boutr = jnp.pad(bout, (0, D_PAD - CLASSES)).reshape(1, D_PAD)
    b11r = b11.reshape(1, HIDDEN)
    zero = jnp.zeros((N_PAD, D_PAD), jnp.float32)

    m_table = _make_table(w11p, woutp)
    partials = _sc_scatter()(m_table, colp, rowp, zero)
    ls, h = _finalize(partials, b11r, woutp, boutr)
    return (ls[:N, :CLASSES], h[:N, :CLASSES])


# speedup vs baseline: 13.3626x; 1.5152x over previous
import functools

import jax
import jax.numpy as jnp
from jax import lax
from jax.experimental import pallas as pl
from jax.experimental.pallas import tpu as pltpu
from jax.experimental.pallas import tpu_sc as plsc

N = 10000
E = 320000
HIDDEN = 128
CLASSES = 40

D_PAD = 48
N_PAD = 10112
NC = 2
NS = 16
LANE = 128
CHUNKS = E // LANE
N_CORE0 = 1668
N_CORE1 = CHUNKS - N_CORE0
Q0, R0 = divmod(N_CORE0, NS)
Q1, R1 = divmod(N_CORE1, NS)
CH_MAX0 = Q0 + (1 if R0 else 0)
CH_MAX1 = Q1 + (1 if R1 else 0)
STRIPE = N_PAD // NS


def _mm_body(w11_ref, wout_ref, m_ref):
    m_ref[...] = lax.dot_general(
        w11_ref[...], wout_ref[...],
        dimension_numbers=(((0,), (1,)), ((), ())),
        preferred_element_type=jnp.float32,
    )


def _make_table(w11p, woutp):
    return pl.pallas_call(
        _mm_body,
        out_shape=jax.ShapeDtypeStruct((N_PAD, D_PAD), jnp.float32),
    )(w11p, woutp)


def _sc_body(m_hbm, ei_hbm, zero_hbm, out_hbm,
             col_v, row_v, buf0, buf1, acc, sem0, sem1):
    cid = lax.axis_index("c")
    sid = lax.axis_index("s")
    pltpu.sync_copy(zero_hbm.at[pl.ds(sid * STRIPE, STRIPE)],
                    acc.at[pl.ds(sid * STRIPE, STRIPE)])
    plsc.subcore_barrier()

    bufs = (buf0, buf1)
    sems = (sem0, sem1)

    def run(ch_max, q, r, base):
        start = base + sid * q + jnp.minimum(sid, r)
        cnt = q + jnp.where(sid < r, 1, 0)
        pltpu.sync_copy(ei_hbm.at[0, pl.ds(start * LANE, ch_max * LANE)],
                        row_v.at[pl.ds(0, ch_max * LANE)])
        pltpu.sync_copy(ei_hbm.at[1, pl.ds(start * LANE, ch_max * LANE)],
                        col_v.at[pl.ds(0, ch_max * LANE)])

        def fire(c, b):
            pltpu.async_copy(
                m_hbm.at[col_v.at[pl.ds(c * LANE, LANE)]], bufs[b], sems[b])

        def drain_scatter(c, b):
            pltpu.make_async_copy(m_hbm.at[pl.ds(0, LANE)],
                                  bufs[b], sems[b]).wait()
            pltpu.sync_copy(bufs[b],
                            acc.at[row_v.at[pl.ds(c * LANE, LANE)]], add=True)

        @pl.when(cnt > 0)
        def _():
            fire(0, 0)

        @pl.when(cnt > 1)
        def _():
            fire(1, 1)

        def body(j, carry):
            for b in range(2):
                c = 2 * j + b

                @pl.when(c < cnt)
                def _():
                    drain_scatter(c, b)

                    @pl.when(c + 2 < cnt)
                    def _():
                        fire(c + 2, b)

            return carry

        lax.fori_loop(0, (cnt + 1) // 2, body, 0)

    @pl.when(cid == 0)
    def _():
        run(CH_MAX0, Q0, R0, 0)

    @pl.when(cid == 1)
    def _():
        run(CH_MAX1, Q1, R1, N_CORE0)

    plsc.subcore_barrier()
    pltpu.sync_copy(acc.at[pl.ds(sid * STRIPE, STRIPE)],
                    out_hbm.at[cid, pl.ds(sid * STRIPE, STRIPE)])


@functools.cache
def _sc_scatter():
    ch_max = max(CH_MAX0, CH_MAX1)
    return pl.kernel(
        _sc_body,
        out_type=jax.ShapeDtypeStruct((NC, N_PAD, D_PAD), jnp.float32),
        mesh=plsc.VectorSubcoreMesh(core_axis_name="c", subcore_axis_name="s",
                                    num_cores=NC, num_subcores=NS),
        scratch_types=[
            pltpu.VMEM((ch_max * LANE,), jnp.int32),
            pltpu.VMEM((ch_max * LANE,), jnp.int32),
            pltpu.VMEM((LANE, D_PAD), jnp.float32),
            pltpu.VMEM((LANE, D_PAD), jnp.float32),
            pltpu.VMEM_SHARED((N_PAD, D_PAD), jnp.float32),
            pltpu.SemaphoreType.DMA,
            pltpu.SemaphoreType.DMA,
        ],
        compiler_params=pltpu.CompilerParams(use_tc_tiling_on_sc=False),
    )


def _softmax_body(p_ref, b11_ref, wout_ref, bout_ref, ls_ref, h_ref):
    bias = lax.dot_general(
        b11_ref[...], wout_ref[...],
        dimension_numbers=(((1,), (1,)), ((), ())),
        preferred_element_type=jnp.float32,
    ) + bout_ref[...]
    h = p_ref[0] + p_ref[1] + bias
    col = lax.broadcasted_iota(jnp.int32, (N_PAD, D_PAD), 1)
    hm = jnp.where(col < CLASSES, h, -jnp.inf)
    m = jnp.max(hm, axis=1, keepdims=True)
    lse = jnp.log(jnp.sum(jnp.exp(hm - m), axis=1, keepdims=True)) + m
    ls = h - lse
    ls_ref[...] = ls[:N, :CLASSES]
    h_ref[...] = h[:N, :CLASSES]


def _finalize(partials, b11r, woutp, boutr):
    return pl.pallas_call(
        _softmax_body,
        out_shape=(jax.ShapeDtypeStruct((N, CLASSES), jnp.float32),
                   jax.ShapeDtypeStruct((N, CLASSES), jnp.float32)),
    )(partials, b11r, woutp, boutr)


def kernel(x, edge_index, W11, b11, Wout, bout):
    del x
    w11p = jnp.pad(W11, ((0, 0), (0, N_PAD - N)))
    woutp = jnp.pad(Wout, ((0, D_PAD - CLASSES), (0, 0)))
    boutr = jnp.pad(bout, (0, D_PAD - CLASSES)).reshape(1, D_PAD)
    b11r = b11.reshape(1, HIDDEN)
    zero = jnp.zeros((N_PAD, D_PAD), jnp.float32)

    m_table = _make_table(w11p, woutp)
    partials = _sc_scatter()(m_table, edge_index, zero)
    return _finalize(partials, b11r, woutp, boutr)


# speedup vs baseline: 15.0143x; 1.1236x over previous
import functools

import jax
import jax.numpy as jnp
from jax import lax
from jax.experimental import pallas as pl
from jax.experimental.pallas import tpu as pltpu
from jax.experimental.pallas import tpu_sc as plsc

N = 10000
E = 320000
HIDDEN = 128
CLASSES = 40

D_PAD = 48
N_PAD = 10112
NC = 2
NS = 16
LANE = 128
CHUNKS = E // LANE
N_CORE0 = 1316
N_CORE1 = CHUNKS - N_CORE0
Q0, R0 = divmod(N_CORE0, NS)
Q1, R1 = divmod(N_CORE1, NS)
CH_MAX0 = Q0 + (1 if R0 else 0)
CH_MAX1 = Q1 + (1 if R1 else 0)
STRIPE = N_PAD // NS


def _mm_body(w11_ref, wout_ref, m_ref):
    m_ref[...] = lax.dot_general(
        w11_ref[...], wout_ref[...],
        dimension_numbers=(((0,), (1,)), ((), ())),
        preferred_element_type=jnp.float32,
    )


def _make_table(w11, woutp):
    return pl.pallas_call(
        _mm_body,
        out_shape=jax.ShapeDtypeStruct((N, D_PAD), jnp.float32),
    )(w11, woutp)


def _sc_body(m_hbm, ei_hbm, zero_hbm, out_hbm,
             col_v, row_v, buf0, buf1, acc, sem0, sem1):
    cid = lax.axis_index("c")
    sid = lax.axis_index("s")
    pltpu.sync_copy(zero_hbm.at[pl.ds(sid * STRIPE, STRIPE)],
                    acc.at[pl.ds(sid * STRIPE, STRIPE)])
    plsc.subcore_barrier()

    bufs = (buf0, buf1)
    sems = (sem0, sem1)

    def run(ch_max, q, r, base):
        start = base + sid * q + jnp.minimum(sid, r)
        cnt = q + jnp.where(sid < r, 1, 0)
        pltpu.sync_copy(ei_hbm.at[0, pl.ds(start * LANE, ch_max * LANE)],
                        row_v.at[pl.ds(0, ch_max * LANE)])
        pltpu.sync_copy(ei_hbm.at[1, pl.ds(start * LANE, ch_max * LANE)],
                        col_v.at[pl.ds(0, ch_max * LANE)])

        def fire(c, b):
            pltpu.async_copy(
                m_hbm.at[col_v.at[pl.ds(c * LANE, LANE)]], bufs[b], sems[b])

        def drain_scatter(c, b):
            pltpu.make_async_copy(m_hbm.at[pl.ds(0, LANE)],
                                  bufs[b], sems[b]).wait()
            pltpu.sync_copy(bufs[b],
                            acc.at[row_v.at[pl.ds(c * LANE, LANE)]], add=True)

        @pl.when(cnt > 0)
        def _():
            fire(0, 0)

        @pl.when(cnt > 1)
        def _():
            fire(1, 1)

        def body(j, carry):
            for b in range(2):
                c = 2 * j + b

                @pl.when(c < cnt)
                def _():
                    drain_scatter(c, b)

                    @pl.when(c + 2 < cnt)
                    def _():
                        fire(c + 2, b)

            return carry

        lax.fori_loop(0, (cnt + 1) // 2, body, 0)

    @pl.when(cid == 0)
    def _():
        run(CH_MAX0, Q0, R0, 0)

    @pl.when(cid == 1)
    def _():
        run(CH_MAX1, Q1, R1, N_CORE0)

    plsc.subcore_barrier()
    pltpu.sync_copy(acc.at[pl.ds(sid * STRIPE, STRIPE)],
                    out_hbm.at[cid, pl.ds(sid * STRIPE, STRIPE)])


@functools.cache
def _sc_scatter():
    ch_max = max(CH_MAX0, CH_MAX1)
    return pl.kernel(
        _sc_body,
        out_type=jax.ShapeDtypeStruct((NC, N_PAD, D_PAD), jnp.float32),
        mesh=plsc.VectorSubcoreMesh(core_axis_name="c", subcore_axis_name="s",
                                    num_cores=NC, num_subcores=NS),
        scratch_types=[
            pltpu.VMEM((ch_max * LANE,), jnp.int32),
            pltpu.VMEM((ch_max * LANE,), jnp.int32),
            pltpu.VMEM((LANE, D_PAD), jnp.float32),
            pltpu.VMEM((LANE, D_PAD), jnp.float32),
            pltpu.VMEM_SHARED((N_PAD, D_PAD), jnp.float32),
            pltpu.SemaphoreType.DMA,
            pltpu.SemaphoreType.DMA,
        ],
        compiler_params=pltpu.CompilerParams(use_tc_tiling_on_sc=False),
    )


def _softmax_body(p_ref, b11_ref, wout_ref, bout_ref, ls_ref, h_ref):
    bias = lax.dot_general(
        b11_ref[...], wout_ref[...],
        dimension_numbers=(((1,), (1,)), ((), ())),
        preferred_element_type=jnp.float32,
    ) + bout_ref[...]
    h = p_ref[0] + p_ref[1] + bias
    col = lax.broadcasted_iota(jnp.int32, (N_PAD, D_PAD), 1)
    hm = jnp.where(col < CLASSES, h, -jnp.inf)
    m = jnp.max(hm, axis=1, keepdims=True)
    lse = jnp.log(jnp.sum(jnp.exp(hm - m), axis=1, keepdims=True)) + m
    ls = h - lse
    ls_ref[...] = ls[:N, :CLASSES]
    h_ref[...] = h[:N, :CLASSES]


def _finalize(partials, b11r, woutp, boutr):
    return pl.pallas_call(
        _softmax_body,
        out_shape=(jax.ShapeDtypeStruct((N, CLASSES), jnp.float32),
                   jax.ShapeDtypeStruct((N, CLASSES), jnp.float32)),
    )(partials, b11r, woutp, boutr)


def kernel(x, edge_index, W11, b11, Wout, bout):
    del x
    woutp = jnp.pad(Wout, ((0, D_PAD - CLASSES), (0, 0)))
    boutr = jnp.pad(bout, (0, D_PAD - CLASSES)).reshape(1, D_PAD)
    b11r = b11.reshape(1, HIDDEN)
    zero = jnp.zeros((N_PAD, D_PAD), jnp.float32)

    m_table = _make_table(W11, woutp)
    partials = _sc_scatter()(m_table, edge_index, zero)
    return _finalize(partials, b11r, woutp, boutr)


# speedup vs baseline: 17.5609x; 1.1696x over previous
import functools

import jax
import jax.numpy as jnp
from jax import lax
from jax.experimental import pallas as pl
from jax.experimental.pallas import tpu as pltpu
from jax.experimental.pallas import tpu_sc as plsc

N = 10000
E = 320000
HIDDEN = 128
CLASSES = 40

D_PAD = 48
N_PAD = 10112
NC = 2
NS = 16
LANE = 128
CHUNKS = E // LANE
N_CORE0 = 1252
N_CORE1 = CHUNKS - N_CORE0
Q0, R0 = divmod(N_CORE0, NS)
Q1, R1 = divmod(N_CORE1, NS)
CH_MAX0 = Q0 + (1 if R0 else 0)
CH_MAX1 = Q1 + (1 if R1 else 0)
STRIPE = N_PAD // NS


def _mm_body(w11_ref, wout_ref, m_ref):
    m_ref[...] = lax.dot_general(
        w11_ref[...], wout_ref[...],
        dimension_numbers=(((0,), (1,)), ((), ())),
        preferred_element_type=jnp.float32,
    )


def _make_table(w11, woutp):
    return pl.pallas_call(
        _mm_body,
        out_shape=jax.ShapeDtypeStruct((N, D_PAD), jnp.float32),
    )(w11, woutp)


NBUF = 4


def _sc_body(m_hbm, ei_hbm, zero_hbm, out_hbm,
             col_v, row_v, buf0, buf1, buf2, buf3, acc,
             gs0, gs1, gs2, gs3, ss0, ss1, ss2, ss3):
    cid = lax.axis_index("c")
    sid = lax.axis_index("s")
    pltpu.sync_copy(zero_hbm.at[pl.ds(sid * STRIPE, STRIPE)],
                    acc.at[pl.ds(sid * STRIPE, STRIPE)])
    plsc.subcore_barrier()

    bufs = (buf0, buf1, buf2, buf3)
    gsem = (gs0, gs1, gs2, gs3)
    ssem = (ss0, ss1, ss2, ss3)

    def run(ch_max, q, r, base):
        start = base + sid * q + jnp.minimum(sid, r)
        cnt = q + jnp.where(sid < r, 1, 0)
        pltpu.sync_copy(ei_hbm.at[0, pl.ds(start * LANE, ch_max * LANE)],
                        row_v.at[pl.ds(0, ch_max * LANE)])
        pltpu.sync_copy(ei_hbm.at[1, pl.ds(start * LANE, ch_max * LANE)],
                        col_v.at[pl.ds(0, ch_max * LANE)])

        def gather_fire(c, b):
            pltpu.async_copy(
                m_hbm.at[col_v.at[pl.ds(c * LANE, LANE)]], bufs[b], gsem[b])

        def gather_wait(b):
            pltpu.make_async_copy(m_hbm.at[pl.ds(0, LANE)],
                                  bufs[b], gsem[b]).wait()

        def scatter_fire(c, b):
            pltpu.async_copy(bufs[b],
                             acc.at[row_v.at[pl.ds(c * LANE, LANE)]],
                             ssem[b], add=True)

        def scatter_wait(b):
            pltpu.make_async_copy(m_hbm.at[pl.ds(0, LANE)],
                                  bufs[b], ssem[b]).wait()

        @pl.when(cnt > 0)
        def _():
            gather_fire(0, 0)

        @pl.when(cnt > 1)
        def _():
            gather_fire(1, 1)

        def body(j, carry):
            for b in range(NBUF):
                c = NBUF * j + b
                bo = (b + 2) % NBUF

                @pl.when(c < cnt)
                def _():
                    @pl.when(c >= 2)
                    def _():
                        scatter_wait(bo)

                    @pl.when(c + 2 < cnt)
                    def _():
                        gather_fire(c + 2, bo)

                    gather_wait(b)
                    scatter_fire(c, b)

            return carry

        lax.fori_loop(0, (cnt + NBUF - 1) // NBUF, body, 0)

        for b in range(NBUF):
            last = (cnt >= 1) & ((cnt - 1) % NBUF == b)
            prev = (cnt >= 2) & ((cnt - 2) % NBUF == b)

            @pl.when(last | prev)
            def _():
                scatter_wait(b)

    @pl.when(cid == 0)
    def _():
        run(CH_MAX0, Q0, R0, 0)

    @pl.when(cid == 1)
    def _():
        run(CH_MAX1, Q1, R1, N_CORE0)

    plsc.subcore_barrier()
    pltpu.sync_copy(acc.at[pl.ds(sid * STRIPE, STRIPE)],
                    out_hbm.at[cid, pl.ds(sid * STRIPE, STRIPE)])


@functools.cache
def _sc_scatter():
    ch_max = max(CH_MAX0, CH_MAX1)
    return pl.kernel(
        _sc_body,
        out_type=jax.ShapeDtypeStruct((NC, N_PAD, D_PAD), jnp.float32),
        mesh=plsc.VectorSubcoreMesh(core_axis_name="c", subcore_axis_name="s",
                                    num_cores=NC, num_subcores=NS),
        scratch_types=[
            pltpu.VMEM((ch_max * LANE,), jnp.int32),
            pltpu.VMEM((ch_max * LANE,), jnp.int32),
            pltpu.VMEM((LANE, D_PAD), jnp.float32),
            pltpu.VMEM((LANE, D_PAD), jnp.float32),
            pltpu.VMEM((LANE, D_PAD), jnp.float32),
            pltpu.VMEM((LANE, D_PAD), jnp.float32),
            pltpu.VMEM_SHARED((N_PAD, D_PAD), jnp.float32),
            pltpu.SemaphoreType.DMA,
            pltpu.SemaphoreType.DMA,
            pltpu.SemaphoreType.DMA,
            pltpu.SemaphoreType.DMA,
            pltpu.SemaphoreType.DMA,
            pltpu.SemaphoreType.DMA,
            pltpu.SemaphoreType.DMA,
            pltpu.SemaphoreType.DMA,
        ],
        compiler_params=pltpu.CompilerParams(use_tc_tiling_on_sc=False),
    )


def _softmax_body(p_ref, b11_ref, wout_ref, bout_ref, ls_ref, h_ref):
    bias = lax.dot_general(
        b11_ref[...], wout_ref[...],
        dimension_numbers=(((1,), (1,)), ((), ())),
        preferred_element_type=jnp.float32,
    ) + bout_ref[...]
    h = p_ref[0] + p_ref[1] + bias
    col = lax.broadcasted_iota(jnp.int32, (N_PAD, D_PAD), 1)
    hm = jnp.where(col < CLASSES, h, -jnp.inf)
    m = jnp.max(hm, axis=1, keepdims=True)
    lse = jnp.log(jnp.sum(jnp.exp(hm - m), axis=1, keepdims=True)) + m
    ls = h - lse
    ls_ref[...] = ls[:N, :CLASSES]
    h_ref[...] = h[:N, :CLASSES]


def _finalize(partials, b11r, woutp, boutr):
    return pl.pallas_call(
        _softmax_body,
        out_shape=(jax.ShapeDtypeStruct((N, CLASSES), jnp.float32),
                   jax.ShapeDtypeStruct((N, CLASSES), jnp.float32)),
    )(partials, b11r, woutp, boutr)


def kernel(x, edge_index, W11, b11, Wout, bout):
    del x
    woutp = jnp.pad(Wout, ((0, D_PAD - CLASSES), (0, 0)))
    boutr = jnp.pad(bout, (0, D_PAD - CLASSES)).reshape(1, D_PAD)
    b11r = b11.reshape(1, HIDDEN)
    zero = jnp.zeros((N_PAD, D_PAD), jnp.float32)

    m_table = _make_table(W11, woutp)
    partials = _sc_scatter()(m_table, edge_index, zero)
    return _finalize(partials, b11r, woutp, boutr)


# speedup vs baseline: 18.1669x; 1.0345x over previous
import functools

import jax
import jax.numpy as jnp
from jax import lax
from jax.experimental import pallas as pl
from jax.experimental.pallas import tpu as pltpu
from jax.experimental.pallas import tpu_sc as plsc

N = 10000
E = 320000
HIDDEN = 128
CLASSES = 40

D_PAD = 48
N_PAD = 10112
NC = 2
NS = 16
LANE = 128
CHUNKS = E // LANE
N_CORE0 = 1252
N_CORE1 = CHUNKS - N_CORE0
Q0, R0 = divmod(N_CORE0, NS)
Q1, R1 = divmod(N_CORE1, NS)
CH_MAX0 = Q0 + (1 if R0 else 0)
CH_MAX1 = Q1 + (1 if R1 else 0)
STRIPE = N_PAD // NS


def _mm_body(w11_ref, wout_ref, m_ref):
    m_ref[...] = lax.dot_general(
        w11_ref[...], wout_ref[...],
        dimension_numbers=(((0,), (1,)), ((), ())),
        preferred_element_type=jnp.float32,
    )


def _make_table(w11, woutp):
    return pl.pallas_call(
        _mm_body,
        out_shape=jax.ShapeDtypeStruct((N, D_PAD), jnp.float32),
    )(w11, woutp)


NBUF = 6


def _sc_body(m_hbm, ei_hbm, zero_hbm, out_hbm,
             col_v, row_v, buf0, buf1, buf2, buf3, buf4, buf5, acc,
             gs0, gs1, gs2, gs3, gs4, gs5, ss0, ss1, ss2, ss3, ss4, ss5):
    cid = lax.axis_index("c")
    sid = lax.axis_index("s")
    pltpu.sync_copy(zero_hbm.at[pl.ds(sid * STRIPE, STRIPE)],
                    acc.at[pl.ds(sid * STRIPE, STRIPE)])
    plsc.subcore_barrier()

    bufs = (buf0, buf1, buf2, buf3, buf4, buf5)
    gsem = (gs0, gs1, gs2, gs3, gs4, gs5)
    ssem = (ss0, ss1, ss2, ss3, ss4, ss5)

    def run(ch_max, q, r, base):
        start = base + sid * q + jnp.minimum(sid, r)
        cnt = q + jnp.where(sid < r, 1, 0)
        pltpu.sync_copy(ei_hbm.at[0, pl.ds(start * LANE, ch_max * LANE)],
                        row_v.at[pl.ds(0, ch_max * LANE)])
        pltpu.sync_copy(ei_hbm.at[1, pl.ds(start * LANE, ch_max * LANE)],
                        col_v.at[pl.ds(0, ch_max * LANE)])

        def gather_fire(c, b):
            pltpu.async_copy(
                m_hbm.at[col_v.at[pl.ds(c * LANE, LANE)]], bufs[b], gsem[b])

        def gather_wait(b):
            pltpu.make_async_copy(m_hbm.at[pl.ds(0, LANE)],
                                  bufs[b], gsem[b]).wait()

        def scatter_fire(c, b):
            pltpu.async_copy(bufs[b],
                             acc.at[row_v.at[pl.ds(c * LANE, LANE)]],
                             ssem[b], add=True)

        def scatter_wait(b):
            pltpu.make_async_copy(m_hbm.at[pl.ds(0, LANE)],
                                  bufs[b], ssem[b]).wait()

        @pl.when(cnt > 0)
        def _():
            gather_fire(0, 0)

        @pl.when(cnt > 1)
        def _():
            gather_fire(1, 1)

        @pl.when(cnt > 2)
        def _():
            gather_fire(2, 2)

        def body(j, carry):
            for b in range(NBUF):
                c = NBUF * j + b
                bo = (b + 3) % NBUF

                @pl.when(c < cnt)
                def _():
                    @pl.when(c >= 3)
                    def _():
                        scatter_wait(bo)

                    @pl.when(c + 3 < cnt)
                    def _():
                        gather_fire(c + 3, bo)

                    gather_wait(b)
                    scatter_fire(c, b)

            return carry

        lax.fori_loop(0, (cnt + NBUF - 1) // NBUF, body, 0)

        for b in range(NBUF):
            outst = ((cnt >= 1) & ((cnt - 1) % NBUF == b)) \
                | ((cnt >= 2) & ((cnt - 2) % NBUF == b)) \
                | ((cnt >= 3) & ((cnt - 3) % NBUF == b))

            @pl.when(outst)
            def _():
                scatter_wait(b)

    @pl.when(cid == 0)
    def _():
        run(CH_MAX0, Q0, R0, 0)

    @pl.when(cid == 1)
    def _():
        run(CH_MAX1, Q1, R1, N_CORE0)

    plsc.subcore_barrier()
    pltpu.sync_copy(acc.at[pl.ds(sid * STRIPE, STRIPE)],
                    out_hbm.at[cid, pl.ds(sid * STRIPE, STRIPE)])


@functools.cache
def _sc_scatter():
    ch_max = max(CH_MAX0, CH_MAX1)
    return pl.kernel(
        _sc_body,
        out_type=jax.ShapeDtypeStruct((NC, N_PAD, D_PAD), jnp.float32),
        mesh=plsc.VectorSubcoreMesh(core_axis_name="c", subcore_axis_name="s",
                                    num_cores=NC, num_subcores=NS),
        scratch_types=[
            pltpu.VMEM((ch_max * LANE,), jnp.int32),
            pltpu.VMEM((ch_max * LANE,), jnp.int32),
            pltpu.VMEM((LANE, D_PAD), jnp.float32),
            pltpu.VMEM((LANE, D_PAD), jnp.float32),
            pltpu.VMEM((LANE, D_PAD), jnp.float32),
            pltpu.VMEM((LANE, D_PAD), jnp.float32),
            pltpu.VMEM((LANE, D_PAD), jnp.float32),
            pltpu.VMEM((LANE, D_PAD), jnp.float32),
            pltpu.VMEM_SHARED((N_PAD, D_PAD), jnp.float32),
        ] + [pltpu.SemaphoreType.DMA] * 12,
        compiler_params=pltpu.CompilerParams(use_tc_tiling_on_sc=False),
    )


def _softmax_body(p_ref, b11_ref, wout_ref, bout_ref, ls_ref, h_ref):
    bias = lax.dot_general(
        b11_ref[...], wout_ref[...],
        dimension_numbers=(((1,), (1,)), ((), ())),
        preferred_element_type=jnp.float32,
    ) + bout_ref[...]
    h = p_ref[0] + p_ref[1] + bias
    col = lax.broadcasted_iota(jnp.int32, (N_PAD, D_PAD), 1)
    hm = jnp.where(col < CLASSES, h, -jnp.inf)
    m = jnp.max(hm, axis=1, keepdims=True)
    lse = jnp.log(jnp.sum(jnp.exp(hm - m), axis=1, keepdims=True)) + m
    ls = h - lse
    ls_ref[...] = ls[:N, :CLASSES]
    h_ref[...] = h[:N, :CLASSES]


def _finalize(partials, b11r, woutp, boutr):
    return pl.pallas_call(
        _softmax_body,
        out_shape=(jax.ShapeDtypeStruct((N, CLASSES), jnp.float32),
                   jax.ShapeDtypeStruct((N, CLASSES), jnp.float32)),
    )(partials, b11r, woutp, boutr)


def kernel(x, edge_index, W11, b11, Wout, bout):
    del x
    woutp = jnp.pad(Wout, ((0, D_PAD - CLASSES), (0, 0)))
    boutr = jnp.pad(bout, (0, D_PAD - CLASSES)).reshape(1, D_PAD)
    b11r = b11.reshape(1, HIDDEN)
    zero = jnp.zeros((N_PAD, D_PAD), jnp.float32)

    m_table = _make_table(W11, woutp)
    partials = _sc_scatter()(m_table, edge_index, zero)
    return _finalize(partials, b11r, woutp, boutr)


# speedup vs baseline: 19.7107x; 1.0850x over previous
import functools

import jax
import jax.numpy as jnp
from jax import lax
from jax.experimental import pallas as pl
from jax.experimental.pallas import tpu as pltpu
from jax.experimental.pallas import tpu_sc as plsc

N = 10000
E = 320000
HIDDEN = 128
CLASSES = 40

D_PAD = 48
N_PAD = 10112
NC = 2
NS = 16
LANE = 128
CHUNKS = E // LANE
N_CORE0 = 1252
N_CORE1 = CHUNKS - N_CORE0
Q0, R0 = divmod(N_CORE0, NS)
Q1, R1 = divmod(N_CORE1, NS)
CH_MAX0 = Q0 + (1 if R0 else 0)
CH_MAX1 = Q1 + (1 if R1 else 0)
STRIPE = N_PAD // NS


def _mm_body(w11_ref, wout_ref, m_ref):
    m_ref[...] = lax.dot_general(
        w11_ref[...], wout_ref[...],
        dimension_numbers=(((0,), (1,)), ((), ())),
        preferred_element_type=jnp.float32,
    )


def _make_table(w11, woutp):
    return pl.pallas_call(
        _mm_body,
        out_shape=jax.ShapeDtypeStruct((N, D_PAD), jnp.float32),
    )(w11, woutp)


NBUF = 6


def _sc_body(m_hbm, ei_hbm, zero_hbm, out_hbm,
             col_v, row_v, buf0, buf1, buf2, buf3, buf4, buf5, acc,
             gs0, gs1, gs2, gs3, gs4, gs5, ss0, ss1, ss2, ss3, ss4, ss5):
    cid = lax.axis_index("c")
    sid = lax.axis_index("s")
    pltpu.sync_copy(zero_hbm.at[pl.ds(sid * STRIPE, STRIPE)],
                    acc.at[pl.ds(sid * STRIPE, STRIPE)])
    plsc.subcore_barrier()

    bufs = (buf0, buf1, buf2, buf3, buf4, buf5)
    gsem = (gs0, gs1, gs2, gs3, gs4, gs5)
    ssem = (ss0, ss1, ss2, ss3, ss4, ss5)

    def run(ch_max, q, r, base):
        start = base + sid * q + jnp.minimum(sid, r)
        cnt = q + jnp.where(sid < r, 1, 0)
        pltpu.sync_copy(ei_hbm.at[0, pl.ds(start * LANE, ch_max * LANE)],
                        row_v.at[pl.ds(0, ch_max * LANE)])
        pltpu.sync_copy(ei_hbm.at[1, pl.ds(start * LANE, ch_max * LANE)],
                        col_v.at[pl.ds(0, ch_max * LANE)])

        def gather_fire(c, b):
            pltpu.async_copy(
                m_hbm.at[col_v.at[pl.ds(c * LANE, LANE)]], bufs[b], gsem[b])

        def gather_wait(b):
            pltpu.make_async_copy(m_hbm.at[pl.ds(0, LANE)],
                                  bufs[b], gsem[b]).wait()

        def scatter_fire(c, b):
            pltpu.async_copy(bufs[b],
                             acc.at[row_v.at[pl.ds(c * LANE, LANE)]],
                             ssem[b], add=True)

        def scatter_wait(b):
            pltpu.make_async_copy(m_hbm.at[pl.ds(0, LANE)],
                                  bufs[b], ssem[b]).wait()

        @pl.when(cnt > 0)
        def _():
            gather_fire(0, 0)

        @pl.when(cnt > 1)
        def _():
            gather_fire(1, 1)

        @pl.when(cnt > 2)
        def _():
            gather_fire(2, 2)

        def body(j, carry):
            for b in range(NBUF):
                c = NBUF * j + b
                bo = (b + 3) % NBUF

                @pl.when(c < cnt)
                def _():
                    @pl.when(c >= 3)
                    def _():
                        scatter_wait(bo)

                    @pl.when(c + 3 < cnt)
                    def _():
                        gather_fire(c + 3, bo)

                    gather_wait(b)
                    scatter_fire(c, b)

            return carry

        lax.fori_loop(0, (cnt + NBUF - 1) // NBUF, body, 0)

        for b in range(NBUF):
            outst = ((cnt >= 1) & ((cnt - 1) % NBUF == b)) \
                | ((cnt >= 2) & ((cnt - 2) % NBUF == b)) \
                | ((cnt >= 3) & ((cnt - 3) % NBUF == b))

            @pl.when(outst)
            def _():
                scatter_wait(b)

    @pl.when(cid == 0)
    def _():
        run(CH_MAX0, Q0, R0, 0)

    @pl.when(cid == 1)
    def _():
        run(CH_MAX1, Q1, R1, N_CORE0)

    plsc.subcore_barrier()
    pltpu.sync_copy(acc.at[pl.ds(sid * STRIPE, STRIPE)],
                    out_hbm.at[cid, pl.ds(sid * STRIPE, STRIPE),
                               pl.ds(0, D_PAD)])


@functools.cache
def _sc_scatter():
    ch_max = max(CH_MAX0, CH_MAX1)
    return pl.kernel(
        _sc_body,
        out_type=jax.ShapeDtypeStruct((NC, N_PAD, 128), jnp.float32),
        mesh=plsc.VectorSubcoreMesh(core_axis_name="c", subcore_axis_name="s",
                                    num_cores=NC, num_subcores=NS),
        scratch_types=[
            pltpu.VMEM((ch_max * LANE,), jnp.int32),
            pltpu.VMEM((ch_max * LANE,), jnp.int32),
            pltpu.VMEM((LANE, D_PAD), jnp.float32),
            pltpu.VMEM((LANE, D_PAD), jnp.float32),
            pltpu.VMEM((LANE, D_PAD), jnp.float32),
            pltpu.VMEM((LANE, D_PAD), jnp.float32),
            pltpu.VMEM((LANE, D_PAD), jnp.float32),
            pltpu.VMEM((LANE, D_PAD), jnp.float32),
            pltpu.VMEM_SHARED((N_PAD, D_PAD), jnp.float32),
        ] + [pltpu.SemaphoreType.DMA] * 12,
        compiler_params=pltpu.CompilerParams(use_tc_tiling_on_sc=False),
    )


def _softmax_body(p_ref, b11_ref, wout_ref, bout_ref, ls_ref, h_ref):
    bias = lax.dot_general(
        b11_ref[...], wout_ref[...],
        dimension_numbers=(((1,), (1,)), ((), ())),
        preferred_element_type=jnp.float32,
    ) + bout_ref[...]
    h = p_ref[0] + p_ref[1] + bias
    col = lax.broadcasted_iota(jnp.int32, (N_PAD, 128), 1)
    hm = jnp.where(col < CLASSES, h, -jnp.inf)
    m = jnp.max(hm, axis=1, keepdims=True)
    lse = jnp.log(jnp.sum(jnp.exp(hm - m), axis=1, keepdims=True)) + m
    ls = h - lse
    ls_ref[...] = ls[:N, :CLASSES]
    h_ref[...] = h[:N, :CLASSES]


def _finalize(partials, b11r, woutp, boutr):
    return pl.pallas_call(
        _softmax_body,
        out_shape=(jax.ShapeDtypeStruct((N, CLASSES), jnp.float32),
                   jax.ShapeDtypeStruct((N, CLASSES), jnp.float32)),
    )(partials, b11r, woutp, boutr)


def kernel(x, edge_index, W11, b11, Wout, bout):
    del x
    woutp = jnp.pad(Wout, ((0, D_PAD - CLASSES), (0, 0)))
    wout128 = jnp.pad(Wout, ((0, 128 - CLASSES), (0, 0)))
    boutr = jnp.pad(bout, (0, 128 - CLASSES)).reshape(1, 128)
    b11r = b11.reshape(1, HIDDEN)
    zero = jnp.zeros((N_PAD, D_PAD), jnp.float32)

    m_table = _make_table(W11, woutp)
    partials = _sc_scatter()(m_table, edge_index, zero)
    return _finalize(partials, b11r, wout128, boutr)
